# baseline (speedup 1.0000x reference)
import functools

import jax
import jax.numpy as jnp
from jax import lax
from jax.experimental import pallas as pl
from jax.experimental.pallas import tpu as pltpu
from jax.experimental.pallas import tpu_sc as plsc

MAX_ATOMIC_NUM = 100
NUM_WYCKOFF = 186
NUM_SG = 230
T_STEPS = 1000
BS = 64
_WY_PAD = 256
_LANES = 16
_MSS_WORKERS = 8


def _sc_gather_body(t_hbm, s_hbm, sgs_hbm, alphas_hbm, acp_hbm, wy_hbm,
                    at_hbm, abs_hbm, abt_hbm, mss_hbm,
                    t_v, s_v, alphas_v, acp_v, at_v, abs_v, abt_v,
                    idx8_v, rows_v, sem):
    wid = lax.axis_index("s") * 2 + lax.axis_index("c")

    @pl.when(wid < _MSS_WORKERS)
    def _():
        base = wid * (BS // _MSS_WORKERS)
        pltpu.sync_copy(sgs_hbm.at[pl.ds(base, BS // _MSS_WORKERS)], idx8_v)
        pltpu.async_copy(wy_hbm.at[idx8_v], rows_v, sem).wait()
        pltpu.sync_copy(rows_v, mss_hbm.at[pl.ds(base, BS // _MSS_WORKERS)])

    @pl.when(wid == _MSS_WORKERS)
    def _():
        pltpu.sync_copy(t_hbm, t_v)
        pltpu.sync_copy(s_hbm, s_v)
        pltpu.sync_copy(alphas_hbm, alphas_v)
        pltpu.sync_copy(acp_hbm, acp_v)
        for c in range(BS // _LANES):
            sl = pl.ds(c * _LANES, _LANES)
            idx_t = t_v[sl]
            idx_s = s_v[sl]
            at_v[sl] = plsc.load_gather(alphas_v, [idx_t])
            abt_v[sl] = plsc.load_gather(acp_v, [idx_t])
            abs_v[sl] = plsc.load_gather(acp_v, [idx_s])
        pltpu.sync_copy(at_v, at_hbm)
        pltpu.sync_copy(abs_v, abs_hbm)
        pltpu.sync_copy(abt_v, abt_hbm)


def _sc_gather(t, s, sgs, alphas, alphas_cumprod, wyckoff):
    mesh = plsc.VectorSubcoreMesh(core_axis_name="c", subcore_axis_name="s")
    rows_per_w = BS // _MSS_WORKERS
    return pl.kernel(
        _sc_gather_body,
        out_type=[
            jax.ShapeDtypeStruct((BS,), jnp.float32),
            jax.ShapeDtypeStruct((BS,), jnp.float32),
            jax.ShapeDtypeStruct((BS,), jnp.float32),
            jax.ShapeDtypeStruct((BS, _WY_PAD), jnp.float32),
        ],
        mesh=mesh,
        compiler_params=pltpu.CompilerParams(needs_layout_passes=False),
        scratch_types=[
            pltpu.VMEM((BS,), jnp.int32),
            pltpu.VMEM((BS,), jnp.int32),
            pltpu.VMEM((T_STEPS,), jnp.float32),
            pltpu.VMEM((T_STEPS,), jnp.float32),
            pltpu.VMEM((BS,), jnp.float32),
            pltpu.VMEM((BS,), jnp.float32),
            pltpu.VMEM((BS,), jnp.float32),
            pltpu.VMEM((rows_per_w,), jnp.int32),
            pltpu.VMEM((rows_per_w, _WY_PAD), jnp.float32),
            pltpu.SemaphoreType.DMA,
        ],
    )(t, s, sgs, alphas, alphas_cumprod, wyckoff)


def _posterior(z, pred, m, a_t, ab_s, ab_t, mask):
    zm = jnp.sum(z * m, axis=-1, keepdims=True)
    left = a_t * z + (1.0 - a_t) * zm
    den = ab_t * z + (1.0 - ab_t) * zm
    den = jnp.where(den == 0.0, 1e-6, den)
    sm = jax.nn.softmax(pred, axis=-1)
    w = sm / den
    W = jnp.sum(w, axis=-1, keepdims=True)
    unnorm = left * (ab_s * w + (1.0 - ab_s) * m * W)
    row = jnp.sum(unnorm, axis=-1, keepdims=True)
    unnorm = jnp.where(row == 0.0, 1e-5, unnorm)
    prob = unnorm / jnp.sum(unnorm, axis=-1, keepdims=True)
    d = prob.shape[-1]
    return jnp.where(mask, prob, 1.0 / d)


def _noise_kernel(z_a_ref, z_ss_ref, pred_a_ref, pred_ss_ref, at_ref, abs_ref,
                  abt_ref, mss_ref, mask_ref, m_a_ref, out_ref):
    a_t = at_ref[:, :][:, :, None]
    ab_s = abs_ref[:, :][:, :, None]
    ab_t = abt_ref[:, :][:, :, None]
    m_ss = mss_ref[:, :NUM_WYCKOFF][:, None, :]
    mask = mask_ref[:, :, :]
    m_a = m_a_ref[:, :][:, None, :]
    prob_a = _posterior(z_a_ref[:, :, :], pred_a_ref[:, :, :], m_a,
                        a_t, ab_s, ab_t, mask)
    prob_ss = _posterior(z_ss_ref[:, :, :], pred_ss_ref[:, :, :], m_ss,
                         a_t, ab_s, ab_t, mask)
    out_ref[:, :, :MAX_ATOMIC_NUM] = prob_a
    out_ref[:, :, MAX_ATOMIC_NUM:] = prob_ss


def kernel(z_t_a, z_t_ss, pred_a, pred_ss, t, s, sgs, node_mask,
           atom_type_marginals, wyckoff_marginals_per_sg, alphas,
           alphas_cumprod):
    B, N, Da = z_t_a.shape
    Dss = z_t_ss.shape[-1]
    wy_pad = jnp.pad(wyckoff_marginals_per_sg, ((0, 0), (0, _WY_PAD - Dss)))
    a_t, ab_s, ab_t, m_ss = _sc_gather(
        t.astype(jnp.int32), s.astype(jnp.int32), sgs.astype(jnp.int32),
        alphas, alphas_cumprod, wy_pad)
    out = pl.pallas_call(
        _noise_kernel,
        out_shape=jax.ShapeDtypeStruct((B, N, Da + Dss), jnp.float32),
    )(
        z_t_a, z_t_ss, pred_a, pred_ss,
        a_t.reshape(B, 1), ab_s.reshape(B, 1), ab_t.reshape(B, 1),
        m_ss,
        node_mask.reshape(B, N, 1),
        atom_type_marginals.reshape(1, Da),
    )
    return out


# speedup vs baseline: 1.9610x; 1.9610x over previous
import jax
import jax.numpy as jnp
from jax.experimental import pallas as pl

MAX_ATOMIC_NUM = 100
NUM_WYCKOFF = 186
NUM_SG = 230
T_STEPS = 1000


def _rowsum(x):
    B, N, D = x.shape
    ones = jnp.full((D, 1), 1.0, dtype=x.dtype)
    r = jnp.dot(x.reshape(B * N, D), ones, preferred_element_type=jnp.float32)
    return r.reshape(B, N, 1)


def _posterior(z, pred, m, a_t, ab_s, ab_t, mask):
    zm = _rowsum(z * m)
    left = a_t * z + (1.0 - a_t) * zm
    den = ab_t * z + (1.0 - ab_t) * zm
    den = jnp.where(den == 0.0, 1e-6, den)
    e = jnp.exp(pred)
    ssum = _rowsum(e)
    w = e / (ssum * den)
    W = _rowsum(w)
    unnorm = left * (ab_s * w + (1.0 - ab_s) * m * W)
    row = _rowsum(unnorm)
    unnorm = jnp.where(row == 0.0, 1e-5, unnorm)
    d = unnorm.shape[-1]
    total = jnp.where(row == 0.0, d * 1e-5, row)
    prob = unnorm * (1.0 / total)
    return jnp.where(mask, prob, 1.0 / d)


def _noise_kernel(z_a_ref, z_ss_ref, pred_a_ref, pred_ss_ref, t_ref, s_ref,
                  sgs_ref, mask_ref, m_a_ref, wy_ref, alphas_ref, acp_ref,
                  out_ref):
    B = z_a_ref.shape[0]

    kt = jax.lax.broadcasted_iota(jnp.int32, (B, T_STEPS), 1)
    oh_t = (t_ref[:, :] == kt).astype(jnp.float32)
    oh_s = (s_ref[:, :] == kt).astype(jnp.float32)
    alphas = alphas_ref[:, :]
    acp = acp_ref[:, :]
    a_t = jnp.sum(oh_t * alphas, axis=1, keepdims=True)
    ab_t = jnp.sum(oh_t * acp, axis=1, keepdims=True)
    ab_s = jnp.sum(oh_s * acp, axis=1, keepdims=True)
    a_t = a_t[:, :, None]
    ab_t = ab_t[:, :, None]
    ab_s = ab_s[:, :, None]

    ksg = jax.lax.broadcasted_iota(jnp.int32, (B, NUM_SG), 1)
    oh_sg = (sgs_ref[:, :] == ksg).astype(jnp.float32)
    m_ss = jnp.dot(oh_sg, wy_ref[:, :],
                   preferred_element_type=jnp.float32)

    mask = mask_ref[:, :, :]
    m_a = m_a_ref[:, :][:, None, :]
    prob_a = _posterior(z_a_ref[:, :, :], pred_a_ref[:, :, :], m_a,
                        a_t, ab_s, ab_t, mask)
    prob_ss = _posterior(z_ss_ref[:, :, :], pred_ss_ref[:, :, :],
                         m_ss[:, None, :], a_t, ab_s, ab_t, mask)
    out_ref[:, :, :MAX_ATOMIC_NUM] = prob_a
    out_ref[:, :, MAX_ATOMIC_NUM:] = prob_ss


def kernel(z_t_a, z_t_ss, pred_a, pred_ss, t, s, sgs, node_mask,
           atom_type_marginals, wyckoff_marginals_per_sg, alphas,
           alphas_cumprod):
    B, N, Da = z_t_a.shape
    Dss = z_t_ss.shape[-1]
    out = pl.pallas_call(
        _noise_kernel,
        out_shape=jax.ShapeDtypeStruct((B, N, Da + Dss), jnp.float32),
    )(
        z_t_a, z_t_ss, pred_a, pred_ss,
        t.astype(jnp.int32).reshape(B, 1),
        s.astype(jnp.int32).reshape(B, 1),
        sgs.astype(jnp.int32).reshape(B, 1),
        node_mask.reshape(B, N, 1),
        atom_type_marginals.reshape(1, Da),
        wyckoff_marginals_per_sg,
        alphas.reshape(1, T_STEPS),
        alphas_cumprod.reshape(1, T_STEPS),
    )
    return out


# speedup vs baseline: 2.1163x; 1.0792x over previous
import jax
import jax.numpy as jnp
from jax.experimental import pallas as pl

MAX_ATOMIC_NUM = 100
NUM_WYCKOFF = 186
NUM_SG = 230
T_STEPS = 1000


def _posterior(z, pred, m, a_t, ab_s, ab_t, node_mask):
    zm = jnp.sum(z * m, axis=-1, keepdims=True)
    left = a_t * z + (1.0 - a_t) * zm
    den = ab_t * z + (1.0 - ab_t) * zm
    den = jnp.where(den == 0.0, 1e-6, den)
    sm = jax.nn.softmax(pred, axis=-1)
    w = sm / den
    W = jnp.sum(w, axis=-1, keepdims=True)
    unnorm = left * (ab_s * w + (1.0 - ab_s) * m * W)
    row = jnp.sum(unnorm, axis=-1, keepdims=True)
    unnorm = jnp.where(row == 0.0, 1e-5, unnorm)
    prob = unnorm / jnp.sum(unnorm, axis=-1, keepdims=True)
    d = prob.shape[-1]
    return jnp.where(node_mask, prob, 1.0 / d)


def _noise_kernel(z_a_ref, z_ss_ref, pred_a_ref, pred_ss_ref, t_ref, s_ref,
                  sgs_ref, mask_ref, m_a_ref, wy_ref, alphas_ref, acp_ref,
                  out_ref):
    B = z_a_ref.shape[0]

    kt = jax.lax.broadcasted_iota(jnp.int32, (B, T_STEPS), 1)
    oh_t = (t_ref[:, :] == kt).astype(jnp.float32)
    oh_s = (s_ref[:, :] == kt).astype(jnp.float32)
    alphas = alphas_ref[:, :]
    acp = acp_ref[:, :]
    a_t = jnp.sum(oh_t * alphas, axis=1, keepdims=True)
    ab_t = jnp.sum(oh_t * acp, axis=1, keepdims=True)
    ab_s = jnp.sum(oh_s * acp, axis=1, keepdims=True)
    a_t = a_t[:, :, None]
    ab_t = ab_t[:, :, None]
    ab_s = ab_s[:, :, None]

    ksg = jax.lax.broadcasted_iota(jnp.int32, (B, NUM_SG), 1)
    oh_sg = (sgs_ref[:, :] == ksg).astype(jnp.float32)
    m_ss = jnp.dot(oh_sg, wy_ref[:, :],
                   preferred_element_type=jnp.float32)

    mask = mask_ref[:, :, :]
    m_a = m_a_ref[:, :][:, None, :]
    prob_a = _posterior(z_a_ref[:, :, :], pred_a_ref[:, :, :], m_a,
                        a_t, ab_s, ab_t, mask)
    prob_ss = _posterior(z_ss_ref[:, :, :], pred_ss_ref[:, :, :],
                         m_ss[:, None, :], a_t, ab_s, ab_t, mask)
    out_ref[:, :, :MAX_ATOMIC_NUM] = prob_a
    out_ref[:, :, MAX_ATOMIC_NUM:] = prob_ss


_GRID = 2


def kernel(z_t_a, z_t_ss, pred_a, pred_ss, t, s, sgs, node_mask,
           atom_type_marginals, wyckoff_marginals_per_sg, alphas,
           alphas_cumprod):
    B, N, Da = z_t_a.shape
    Dss = z_t_ss.shape[-1]
    BB = B // _GRID

    def b3(d):
        return pl.BlockSpec((BB, N, d), lambda i: (i, 0, 0))

    def full(shape):
        return pl.BlockSpec(shape, lambda i: tuple(0 for _ in shape))

    idx_spec = pl.BlockSpec((BB, 1), lambda i: (i, 0))
    out = pl.pallas_call(
        _noise_kernel,
        grid=(_GRID,),
        in_specs=[
            b3(Da), b3(Dss), b3(Da), b3(Dss),
            idx_spec, idx_spec, idx_spec,
            pl.BlockSpec((BB, N, 1), lambda i: (i, 0, 0)),
            full((1, Da)), full((NUM_SG, NUM_WYCKOFF)),
            full((1, T_STEPS)), full((1, T_STEPS)),
        ],
        out_specs=b3(Da + Dss),
        out_shape=jax.ShapeDtypeStruct((B, N, Da + Dss), jnp.float32),
    )(
        z_t_a, z_t_ss, pred_a, pred_ss,
        t.astype(jnp.int32).reshape(B, 1),
        s.astype(jnp.int32).reshape(B, 1),
        sgs.astype(jnp.int32).reshape(B, 1),
        node_mask.reshape(B, N, 1),
        atom_type_marginals.reshape(1, Da),
        wyckoff_marginals_per_sg,
        alphas.reshape(1, T_STEPS),
        alphas_cumprod.reshape(1, T_STEPS),
    )
    return out
